# Initial kernel scaffold; baseline (speedup 1.0000x reference)
import jax
import jax.numpy as jnp
from jax.experimental import pallas as pl


def kernel(x, edge_index, edge_attr, Wn1, bn1, Wn2, bn2, Wc0, bc0, Wc1, bc1, Wc2, bc2, Wo1, bo1, Wo2, bo2, Wo3, bo3):
    raise NotImplementedError("write your pallas kernel here")


# speedup vs baseline: 12.5882x; 12.5882x over previous
import functools

import jax
import jax.numpy as jnp
from jax import lax
from jax.experimental import pallas as pl
from jax.experimental.pallas import tpu as pltpu
from jax.experimental.pallas import tpu_sc as plsc

N = 50000
NP = 50048
E = 800000
EP = 819200
H = 64
HH = 32
ZR = NP // 16
LCH = EP // 16 // 128
DCH = EP // 32 // 128
SLAB = 40
BR = 2000
NB = N // BR


def _sc_mesh():
    return plsc.VectorSubcoreMesh(core_axis_name="c", subcore_axis_name="s",
                                  num_cores=2)


def _make_deg_kernel():
    @functools.partial(
        pl.kernel,
        out_type=jax.ShapeDtypeStruct((2, NP, 16), jnp.float32),
        mesh=_sc_mesh(),
        compiler_params=pltpu.CompilerParams(use_tc_tiling_on_sc=False),
        scratch_types=[
            pltpu.VMEM((SLAB, 128), jnp.int32),
            pltpu.VMEM((128, 16), jnp.float32),
            pltpu.VMEM_SHARED((NP, 16), jnp.float32),
        ],
    )
    def deg_kernel(dst_hbm, zeros_hbm, ones_hbm, out_hbm, dstbuf, ones_v, acc):
        c = lax.axis_index("c")
        s = lax.axis_index("s")
        pltpu.sync_copy(ones_hbm, ones_v)
        pltpu.sync_copy(zeros_hbm, acc.at[pl.ds(s * ZR, ZR)])
        plsc.subcore_barrier()

        def slab_body(k, carry):
            pltpu.sync_copy(dst_hbm.at[c, s, pl.ds(k * SLAB, SLAB)], dstbuf)

            def chunk_body(j, carry2):
                pltpu.sync_copy(ones_v, acc.at[dstbuf.at[j]], add=True)
                return carry2

            return lax.fori_loop(0, SLAB, chunk_body, carry)

        lax.fori_loop(0, DCH // SLAB, slab_body, 0)
        plsc.subcore_barrier()
        pltpu.sync_copy(acc.at[pl.ds(s * ZR, ZR)],
                        out_hbm.at[c, pl.ds(s * ZR, ZR)])

    return deg_kernel


def _make_gs_kernel():
    @functools.partial(
        pl.kernel,
        out_type=jax.ShapeDtypeStruct((2, NP, HH), jnp.float32),
        mesh=_sc_mesh(),
        compiler_params=pltpu.CompilerParams(use_tc_tiling_on_sc=False),
        scratch_types=[
            pltpu.VMEM((128,), jnp.int32),
            pltpu.VMEM((SLAB, 128), jnp.int32),
            pltpu.VMEM((128, HH), jnp.float32),
            pltpu.SemaphoreType.DMA,
            pltpu.VMEM_SHARED((NP, HH), jnp.float32),
        ],
    )
    def gs_kernel(src_hbm, dst_hbm, hs_hbm, zeros_hbm, out_hbm,
                  idxbuf, dstbuf, rows, gsem, acc):
        c = lax.axis_index("c")
        s = lax.axis_index("s")
        tile_base = (c * 16 + s) * (LCH * 128)
        pltpu.sync_copy(zeros_hbm, acc.at[pl.ds(s * ZR, ZR)])
        plsc.subcore_barrier()

        def slab_body(k, carry):
            pltpu.sync_copy(dst_hbm.at[s, pl.ds(k * SLAB, SLAB)], dstbuf)

            def chunk_body(j, carry2):
                base = tile_base + (k * SLAB + j) * 128
                pltpu.sync_copy(src_hbm.at[pl.ds(base, 128)], idxbuf)
                pltpu.async_copy(hs_hbm.at[idxbuf], rows, gsem).wait()
                pltpu.sync_copy(rows, acc.at[dstbuf.at[j]], add=True)
                return carry2

            return lax.fori_loop(0, SLAB, chunk_body, carry)

        lax.fori_loop(0, LCH // SLAB, slab_body, 0)
        plsc.subcore_barrier()
        pltpu.sync_copy(acc.at[pl.ds(s * ZR, ZR)],
                        out_hbm.at[c, pl.ds(s * ZR, ZR)])

    return gs_kernel


def _enc_body(x_ref, p_ref, Wn1_ref, bn1_ref, Wn2_ref, bn2_ref, Wc_ref,
              h_ref, dinv_ref, hs_ref):
    x = x_ref[...]
    h1 = jnp.maximum(jnp.dot(x, Wn1_ref[...],
                             preferred_element_type=jnp.float32)
                     + bn1_ref[...], 0.0)
    h = jnp.maximum(jnp.dot(h1, Wn2_ref[...],
                            preferred_element_type=jnp.float32)
                    + bn2_ref[...], 0.0)
    deg = 1.0 + p_ref[0, :, 0:1] + p_ref[1, :, 0:1]
    dinv = lax.rsqrt(deg)
    h_ref[...] = h
    dinv_ref[...] = dinv
    hs = jnp.dot(h, Wc_ref[...], preferred_element_type=jnp.float32) * dinv
    hs_ref[0] = hs[:, :HH]
    hs_ref[1] = hs[:, HH:]


def _enc_call(x, p, Wn1, bn1, Wn2, bn2, Wc):
    return pl.pallas_call(
        _enc_body,
        grid=(NB,),
        in_specs=[
            pl.BlockSpec((BR, 4), lambda j: (j, 0)),
            pl.BlockSpec((2, BR, 16), lambda j: (0, j, 0)),
            pl.BlockSpec((4, H), lambda j: (0, 0)),
            pl.BlockSpec((1, H), lambda j: (0, 0)),
            pl.BlockSpec((H, H), lambda j: (0, 0)),
            pl.BlockSpec((1, H), lambda j: (0, 0)),
            pl.BlockSpec((H, H), lambda j: (0, 0)),
        ],
        out_specs=[
            pl.BlockSpec((BR, H), lambda j: (j, 0)),
            pl.BlockSpec((BR, 1), lambda j: (j, 0)),
            pl.BlockSpec((2, BR, HH), lambda j: (0, j, 0)),
        ],
        out_shape=[
            jax.ShapeDtypeStruct((N, H), jnp.float32),
            jax.ShapeDtypeStruct((N, 1), jnp.float32),
            jax.ShapeDtypeStruct((2, N, HH), jnp.float32),
        ],
    )(x, p, Wn1, bn1, Wn2, bn2, Wc)


def _mid_body(acc_ref, hs_ref, h_ref, dinv_ref, b_ref, W_ref,
              h2_ref, hs2_ref):
    dinv = dinv_ref[...]
    m = jnp.concatenate([acc_ref[0] + hs_ref[0], acc_ref[1] + hs_ref[1]],
                        axis=1)
    h2 = h_ref[...] + jnp.maximum(m * dinv + b_ref[...], 0.0)
    h2_ref[...] = h2
    hs2 = jnp.dot(h2, W_ref[...], preferred_element_type=jnp.float32) * dinv
    hs2_ref[0] = hs2[:, :HH]
    hs2_ref[1] = hs2[:, HH:]


def _mid_call(acc, hs, h, dinv, b, W):
    return pl.pallas_call(
        _mid_body,
        grid=(NB,),
        in_specs=[
            pl.BlockSpec((2, BR, HH), lambda j: (0, j, 0)),
            pl.BlockSpec((2, BR, HH), lambda j: (0, j, 0)),
            pl.BlockSpec((BR, H), lambda j: (j, 0)),
            pl.BlockSpec((BR, 1), lambda j: (j, 0)),
            pl.BlockSpec((1, H), lambda j: (0, 0)),
            pl.BlockSpec((H, H), lambda j: (0, 0)),
        ],
        out_specs=[
            pl.BlockSpec((BR, H), lambda j: (j, 0)),
            pl.BlockSpec((2, BR, HH), lambda j: (0, j, 0)),
        ],
        out_shape=[
            jax.ShapeDtypeStruct((N, H), jnp.float32),
            jax.ShapeDtypeStruct((2, N, HH), jnp.float32),
        ],
    )(acc, hs, h, dinv, b, W)


def _post_body(acc_ref, hs_ref, h_ref, dinv_ref, b_ref, h3_ref):
    m = jnp.concatenate([acc_ref[0] + hs_ref[0], acc_ref[1] + hs_ref[1]],
                        axis=1)
    h3_ref[...] = h_ref[...] + jnp.maximum(m * dinv_ref[...] + b_ref[...], 0.0)


def _post_call(acc, hs, h, dinv, b):
    return pl.pallas_call(
        _post_body,
        grid=(NB,),
        in_specs=[
            pl.BlockSpec((2, BR, HH), lambda j: (0, j, 0)),
            pl.BlockSpec((2, BR, HH), lambda j: (0, j, 0)),
            pl.BlockSpec((BR, H), lambda j: (j, 0)),
            pl.BlockSpec((BR, 1), lambda j: (j, 0)),
            pl.BlockSpec((1, H), lambda j: (0, 0)),
        ],
        out_specs=pl.BlockSpec((BR, H), lambda j: (j, 0)),
        out_shape=jax.ShapeDtypeStruct((N, H), jnp.float32),
    )(acc, hs, h, dinv, b)


def kernel(x, edge_index, edge_attr, Wn1, bn1, Wn2, bn2, Wc0, bc0, Wc1, bc1,
           Wc2, bc2, Wo1, bo1, Wo2, bo2, Wo3, bo3):
    f32 = jnp.float32
    src = edge_index[0]
    dst = edge_index[1]
    pad = EP - E
    ar = jnp.arange(pad, dtype=jnp.int32)
    srcp = jnp.concatenate([src, (ar * 977) % N])
    dstp = jnp.concatenate([dst, N + (ar % (NP - N))])
    src4 = jnp.concatenate([srcp, srcp + N])
    dst3 = dstp.reshape(16, LCH, 128)
    dstd = dstp.reshape(2, 16, DCH, 128)
    zeros16 = jnp.zeros((ZR, 16), f32)
    zeros32 = jnp.zeros((ZR, HH), f32)
    ones16 = jnp.ones((128, 16), f32)

    deg_k = _make_deg_kernel()
    gs_k = _make_gs_kernel()

    p = deg_k(dstd, zeros16, ones16)

    h, dinv, hs = _enc_call(x, p, Wn1, bn1.reshape(1, H), Wn2,
                            bn2.reshape(1, H), Wc0)

    acc = gs_k(src4, dst3, hs.reshape(2 * N, HH), zeros32)
    h, hs = _mid_call(acc, hs, h, dinv, bc0.reshape(1, H), Wc1)

    acc = gs_k(src4, dst3, hs.reshape(2 * N, HH), zeros32)
    h, hs = _mid_call(acc, hs, h, dinv, bc1.reshape(1, H), Wc2)

    acc = gs_k(src4, dst3, hs.reshape(2 * N, HH), zeros32)
    h3 = _post_call(acc, hs, h, dinv, bc2.reshape(1, H))

    s_mask = x[:, 2] == 1.0
    t_mask = x[:, 3] == 1.0
    has_both = jnp.logical_and(jnp.any(s_mask), jnp.any(t_mask))
    si = jnp.where(has_both, jnp.argmax(s_mask), 0)
    ti = jnp.where(has_both, jnp.argmax(t_mask), N - 1)
    combined = jnp.concatenate([h3[si], h3[ti]], axis=0)
    o = jax.nn.relu(combined @ Wo1 + bo1)
    o = jax.nn.relu(o @ Wo2 + bo2)
    o = o @ Wo3 + bo3
    return o


# speedup vs baseline: 20.2138x; 1.6058x over previous
import functools

import jax
import jax.numpy as jnp
from jax import lax
from jax.experimental import pallas as pl
from jax.experimental.pallas import tpu as pltpu
from jax.experimental.pallas import tpu_sc as plsc

N = 50000
NP = 50048
E = 800000
EP = 819200
H = 64
HH = 32
ZR = NP // 16
LCH = EP // 16 // 128
DCH = EP // 32 // 128
SLAB = 40
BR = 2000
NB = N // BR


def _sc_mesh():
    return plsc.VectorSubcoreMesh(core_axis_name="c", subcore_axis_name="s",
                                  num_cores=2)


def _make_deg_kernel():
    @functools.partial(
        pl.kernel,
        out_type=jax.ShapeDtypeStruct((2, NP, 16), jnp.float32),
        mesh=_sc_mesh(),
        compiler_params=pltpu.CompilerParams(use_tc_tiling_on_sc=False),
        scratch_types=[
            pltpu.VMEM((SLAB, 128), jnp.int32),
            pltpu.VMEM((128, 16), jnp.float32),
            pltpu.VMEM_SHARED((NP, 16), jnp.float32),
        ],
    )
    def deg_kernel(dst_hbm, zeros_hbm, ones_hbm, out_hbm, dstbuf, ones_v, acc):
        c = lax.axis_index("c")
        s = lax.axis_index("s")
        pltpu.sync_copy(ones_hbm, ones_v)
        pltpu.sync_copy(zeros_hbm, acc.at[pl.ds(s * ZR, ZR)])
        plsc.subcore_barrier()

        def slab_body(k, carry):
            pltpu.sync_copy(dst_hbm.at[c, s, pl.ds(k * SLAB, SLAB)], dstbuf)

            def chunk_body(j, carry2):
                pltpu.sync_copy(ones_v, acc.at[dstbuf.at[j]], add=True)
                return carry2

            return lax.fori_loop(0, SLAB, chunk_body, carry)

        lax.fori_loop(0, DCH // SLAB, slab_body, 0)
        plsc.subcore_barrier()
        pltpu.sync_copy(acc.at[pl.ds(s * ZR, ZR)],
                        out_hbm.at[c, pl.ds(s * ZR, ZR)])

    return deg_kernel


QD = 4


def _make_gs_kernel():
    @functools.partial(
        pl.kernel,
        out_type=jax.ShapeDtypeStruct((2, NP, HH), jnp.float32),
        mesh=_sc_mesh(),
        compiler_params=pltpu.CompilerParams(use_tc_tiling_on_sc=False),
        scratch_types=(
            [pltpu.VMEM((128,), jnp.int32)] * QD
            + [pltpu.VMEM((SLAB, 128), jnp.int32)]
            + [pltpu.VMEM((128, HH), jnp.float32)] * QD
            + [pltpu.SemaphoreType.DMA] * (2 * QD)
            + [pltpu.VMEM_SHARED((NP, HH), jnp.float32)]
        ),
    )
    def gs_kernel(src_hbm, dst_hbm, hs_hbm, zeros_hbm, out_hbm, *refs):
        idxs = refs[0:QD]
        dstbuf = refs[QD]
        rows = refs[QD + 1:2 * QD + 1]
        gsems = refs[2 * QD + 1:3 * QD + 1]
        ssems = refs[3 * QD + 1:4 * QD + 1]
        acc = refs[4 * QD + 1]
        c = lax.axis_index("c")
        s = lax.axis_index("s")
        tile_base = (c * 16 + s) * (LCH * 128)
        pltpu.sync_copy(zeros_hbm, acc.at[pl.ds(s * ZR, ZR)])
        plsc.subcore_barrier()

        def slab_body(k, carry):
            pltpu.sync_copy(dst_hbm.at[s, pl.ds(k * SLAB, SLAB)], dstbuf)

            def group_body(g, carry2):
                not_first = jnp.logical_or(k > 0, g > 0)
                for p in range(QD):
                    jj = g * QD + p
                    base = tile_base + (k * SLAB + jj) * 128

                    @pl.when(not_first)
                    def _():
                        pltpu.make_async_copy(
                            rows[p], acc.at[dstbuf.at[jj]], ssems[p]).wait()

                    pltpu.sync_copy(src_hbm.at[pl.ds(base, 128)], idxs[p])
                    pltpu.async_copy(hs_hbm.at[idxs[p]], rows[p], gsems[p])
                for p in range(QD):
                    jj = g * QD + p
                    pltpu.make_async_copy(hs_hbm.at[idxs[p]], rows[p],
                                          gsems[p]).wait()
                    pltpu.async_copy(rows[p], acc.at[dstbuf.at[jj]],
                                     ssems[p], add=True)
                return carry2

            return lax.fori_loop(0, SLAB // QD, group_body, carry)

        lax.fori_loop(0, LCH // SLAB, slab_body, 0)
        for p in range(QD):
            pltpu.make_async_copy(rows[p], acc.at[dstbuf.at[SLAB - QD + p]],
                                  ssems[p]).wait()
        plsc.subcore_barrier()
        pltpu.sync_copy(acc.at[pl.ds(s * ZR, ZR)],
                        out_hbm.at[c, pl.ds(s * ZR, ZR)])

    return gs_kernel


def _enc_body(x_ref, p_ref, Wn1_ref, bn1_ref, Wn2_ref, bn2_ref, Wc_ref,
              h_ref, dinv_ref, hs_ref, st_ref, st_acc):
    j = pl.program_id(0)
    x = x_ref[...]
    h1 = jnp.maximum(jnp.dot(x, Wn1_ref[...],
                             preferred_element_type=jnp.float32)
                     + bn1_ref[...], 0.0)
    h = jnp.maximum(jnp.dot(h1, Wn2_ref[...],
                            preferred_element_type=jnp.float32)
                    + bn2_ref[...], 0.0)
    deg = 1.0 + p_ref[0, :, 0:1] + p_ref[1, :, 0:1]
    dinv = lax.rsqrt(deg)
    h_ref[...] = h
    dinv_ref[...] = dinv
    hs = jnp.dot(h, Wc_ref[...], preferred_element_type=jnp.float32) * dinv
    hs_ref[0] = hs[:, :HH]
    hs_ref[1] = hs[:, HH:]
    rid = lax.broadcasted_iota(jnp.int32, (BR, 1), 0) + j * BR
    ms = jnp.min(jnp.where(x[:, 2:3] == 1.0, rid, N))
    mt = jnp.min(jnp.where(x[:, 3:4] == 1.0, rid, N))

    @pl.when(j == 0)
    def _():
        st_acc[0] = N
        st_acc[1] = N

    st_acc[0] = jnp.minimum(st_acc[0], ms)
    st_acc[1] = jnp.minimum(st_acc[1], mt)

    @pl.when(j == NB - 1)
    def _():
        lane = lax.broadcasted_iota(jnp.int32, (1, 8), 1)
        st_ref[...] = jnp.where(lane == 0, st_acc[0],
                                jnp.where(lane == 1, st_acc[1], 0))


def _enc_call(x, p, Wn1, bn1, Wn2, bn2, Wc):
    return pl.pallas_call(
        _enc_body,
        grid=(NB,),
        in_specs=[
            pl.BlockSpec((BR, 4), lambda j: (j, 0)),
            pl.BlockSpec((2, BR, 16), lambda j: (0, j, 0)),
            pl.BlockSpec((4, H), lambda j: (0, 0)),
            pl.BlockSpec((1, H), lambda j: (0, 0)),
            pl.BlockSpec((H, H), lambda j: (0, 0)),
            pl.BlockSpec((1, H), lambda j: (0, 0)),
            pl.BlockSpec((H, H), lambda j: (0, 0)),
        ],
        out_specs=[
            pl.BlockSpec((BR, H), lambda j: (j, 0)),
            pl.BlockSpec((BR, 1), lambda j: (j, 0)),
            pl.BlockSpec((2, BR, HH), lambda j: (0, j, 0)),
            pl.BlockSpec((1, 8), lambda j: (0, 0)),
        ],
        out_shape=[
            jax.ShapeDtypeStruct((N, H), jnp.float32),
            jax.ShapeDtypeStruct((N, 1), jnp.float32),
            jax.ShapeDtypeStruct((2, N, HH), jnp.float32),
            jax.ShapeDtypeStruct((1, 8), jnp.int32),
        ],
        scratch_shapes=[pltpu.SMEM((2,), jnp.int32)],
    )(x, p, Wn1, bn1, Wn2, bn2, Wc)


def _mid_body(acc_ref, hs_ref, h_ref, dinv_ref, b_ref, W_ref,
              h2_ref, hs2_ref):
    dinv = dinv_ref[...]
    m = jnp.concatenate([acc_ref[0] + hs_ref[0], acc_ref[1] + hs_ref[1]],
                        axis=1)
    h2 = h_ref[...] + jnp.maximum(m * dinv + b_ref[...], 0.0)
    h2_ref[...] = h2
    hs2 = jnp.dot(h2, W_ref[...], preferred_element_type=jnp.float32) * dinv
    hs2_ref[0] = hs2[:, :HH]
    hs2_ref[1] = hs2[:, HH:]


def _mid_call(acc, hs, h, dinv, b, W):
    return pl.pallas_call(
        _mid_body,
        grid=(NB,),
        in_specs=[
            pl.BlockSpec((2, BR, HH), lambda j: (0, j, 0)),
            pl.BlockSpec((2, BR, HH), lambda j: (0, j, 0)),
            pl.BlockSpec((BR, H), lambda j: (j, 0)),
            pl.BlockSpec((BR, 1), lambda j: (j, 0)),
            pl.BlockSpec((1, H), lambda j: (0, 0)),
            pl.BlockSpec((H, H), lambda j: (0, 0)),
        ],
        out_specs=[
            pl.BlockSpec((BR, H), lambda j: (j, 0)),
            pl.BlockSpec((2, BR, HH), lambda j: (0, j, 0)),
        ],
        out_shape=[
            jax.ShapeDtypeStruct((N, H), jnp.float32),
            jax.ShapeDtypeStruct((2, N, HH), jnp.float32),
        ],
    )(acc, hs, h, dinv, b, W)


def _post_body(si_ti_ref, acc_ref, hs_ref, h_ref, dinv_ref, b_ref,
               Wo1_ref, bo1_ref, Wo2_ref, bo2_ref, Wo3_ref, bo3_ref,
               o_ref, sel_acc):
    j = pl.program_id(0)
    m = jnp.concatenate([acc_ref[0] + hs_ref[0], acc_ref[1] + hs_ref[1]],
                        axis=1)
    h3 = h_ref[...] + jnp.maximum(m * dinv_ref[...] + b_ref[...], 0.0)
    rid = lax.broadcasted_iota(jnp.int32, (BR, 1), 0) + j * BR
    sel2 = jnp.concatenate(
        [jnp.where(rid == si_ti_ref[0], 1.0, 0.0),
         jnp.where(rid == si_ti_ref[1], 1.0, 0.0)], axis=1)
    part = lax.dot_general(sel2, h3, (((0,), (0,)), ((), ())),
                           preferred_element_type=jnp.float32)

    @pl.when(j == 0)
    def _():
        sel_acc[...] = jnp.zeros_like(sel_acc)

    sel_acc[...] += part

    @pl.when(j == NB - 1)
    def _():
        combined = jnp.concatenate([sel_acc[0:1, :], sel_acc[1:2, :]], axis=1)
        o = jnp.maximum(jnp.dot(combined, Wo1_ref[...],
                                preferred_element_type=jnp.float32)
                        + bo1_ref[...], 0.0)
        o = jnp.maximum(jnp.dot(o, Wo2_ref[...],
                                preferred_element_type=jnp.float32)
                        + bo2_ref[...], 0.0)
        o_ref[...] = (jnp.dot(o, Wo3_ref[...],
                              preferred_element_type=jnp.float32)
                      + bo3_ref[...])


def _post_call(si_ti, acc, hs, h, dinv, b, Wo1, bo1, Wo2, bo2, Wo3, bo3):
    full = lambda j, *_: (0, 0)
    return pl.pallas_call(
        _post_body,
        grid_spec=pltpu.PrefetchScalarGridSpec(
            num_scalar_prefetch=1,
            grid=(NB,),
            in_specs=[
                pl.BlockSpec((2, BR, HH), lambda j, *_: (0, j, 0)),
                pl.BlockSpec((2, BR, HH), lambda j, *_: (0, j, 0)),
                pl.BlockSpec((BR, H), lambda j, *_: (j, 0)),
                pl.BlockSpec((BR, 1), lambda j, *_: (j, 0)),
                pl.BlockSpec((1, H), full),
                pl.BlockSpec((2 * H, H), full),
                pl.BlockSpec((1, H), full),
                pl.BlockSpec((H, H // 2), full),
                pl.BlockSpec((1, H // 2), full),
                pl.BlockSpec((H // 2, 1), full),
                pl.BlockSpec((1, 1), full),
            ],
            out_specs=pl.BlockSpec((1, 1), full),
            scratch_shapes=[pltpu.VMEM((2, H), jnp.float32)],
        ),
        out_shape=jax.ShapeDtypeStruct((1, 1), jnp.float32),
    )(si_ti, acc, hs, h, dinv, b, Wo1, bo1, Wo2, bo2, Wo3, bo3)


def kernel(x, edge_index, edge_attr, Wn1, bn1, Wn2, bn2, Wc0, bc0, Wc1, bc1,
           Wc2, bc2, Wo1, bo1, Wo2, bo2, Wo3, bo3):
    f32 = jnp.float32
    src = edge_index[0]
    dst = edge_index[1]
    pad = EP - E
    ar = jnp.arange(pad, dtype=jnp.int32)
    srcp = jnp.concatenate([src, (ar * 977) % N])
    dstp = jnp.concatenate([dst, N + (ar % (NP - N))])
    src4 = jnp.concatenate([srcp, srcp + N])
    dst3 = dstp.reshape(16, LCH, 128)
    dstd = dstp.reshape(2, 16, DCH, 128)
    zeros16 = jnp.zeros((ZR, 16), f32)
    zeros32 = jnp.zeros((ZR, HH), f32)
    ones16 = jnp.ones((128, 16), f32)

    deg_k = _make_deg_kernel()
    gs_k = _make_gs_kernel()

    p = deg_k(dstd, zeros16, ones16)

    h, dinv, hs, st = _enc_call(x, p, Wn1, bn1.reshape(1, H), Wn2,
                                bn2.reshape(1, H), Wc0)

    acc = gs_k(src4, dst3, hs.reshape(2 * N, HH), zeros32)
    h, hs = _mid_call(acc, hs, h, dinv, bc0.reshape(1, H), Wc1)

    acc = gs_k(src4, dst3, hs.reshape(2 * N, HH), zeros32)
    h, hs = _mid_call(acc, hs, h, dinv, bc1.reshape(1, H), Wc2)

    acc = gs_k(src4, dst3, hs.reshape(2 * N, HH), zeros32)

    ms, mt = st[0, 0], st[0, 1]
    has_both = jnp.logical_and(ms < N, mt < N)
    si = jnp.where(has_both, ms, 0)
    ti = jnp.where(has_both, mt, N - 1)
    si_ti = jnp.stack([si, ti])

    o = _post_call(si_ti, acc, hs, h, dinv, bc2.reshape(1, H),
                   Wo1, bo1.reshape(1, H), Wo2, bo2.reshape(1, H // 2),
                   Wo3, bo3.reshape(1, 1))
    return o.reshape(1)


# speedup vs baseline: 26.1747x; 1.2949x over previous
import functools

import jax
import jax.numpy as jnp
from jax import lax
from jax.experimental import pallas as pl
from jax.experimental.pallas import tpu as pltpu
from jax.experimental.pallas import tpu_sc as plsc

N = 50000
NP = 50048
E = 800000
EP = 819200
H = 64
HH = 32
ZR = NP // 16
LCH = EP // 16 // 128
DCH = EP // 32 // 128
SLAB = 40
BR = 2000
NB = N // BR


def _sc_mesh():
    return plsc.VectorSubcoreMesh(core_axis_name="c", subcore_axis_name="s",
                                  num_cores=2)


def _make_deg_kernel():
    @functools.partial(
        pl.kernel,
        out_type=jax.ShapeDtypeStruct((2, NP, 16), jnp.float32),
        mesh=_sc_mesh(),
        compiler_params=pltpu.CompilerParams(use_tc_tiling_on_sc=False),
        scratch_types=[
            pltpu.VMEM((SLAB, 128), jnp.int32),
            pltpu.VMEM((128, 16), jnp.float32),
            pltpu.SemaphoreType.DMA,
            pltpu.SemaphoreType.DMA,
            pltpu.SemaphoreType.DMA,
            pltpu.SemaphoreType.DMA,
            pltpu.VMEM_SHARED((NP, 16), jnp.float32),
        ],
    )
    def deg_kernel(dst_hbm, zeros_hbm, ones_hbm, out_hbm, dstbuf, ones_v,
                   s0, s1, s2, s3, acc):
        ssems = (s0, s1, s2, s3)
        c = lax.axis_index("c")
        s = lax.axis_index("s")
        pltpu.sync_copy(ones_hbm, ones_v)
        pltpu.sync_copy(zeros_hbm, acc.at[pl.ds(s * ZR, ZR)])
        plsc.subcore_barrier()

        def slab_body(k, carry):
            pltpu.sync_copy(dst_hbm.at[c, s, pl.ds(k * SLAB, SLAB)], dstbuf)

            def group_body(g, carry2):
                not_first = jnp.logical_or(k > 0, g > 0)
                for p in range(4):
                    jj = g * 4 + p

                    @pl.when(not_first)
                    def _():
                        pltpu.make_async_copy(
                            ones_v, acc.at[dstbuf.at[jj]], ssems[p]).wait()

                    pltpu.async_copy(ones_v, acc.at[dstbuf.at[jj]],
                                     ssems[p], add=True)
                return carry2

            return lax.fori_loop(0, SLAB // 4, group_body, carry)

        lax.fori_loop(0, DCH // SLAB, slab_body, 0)
        for p in range(4):
            pltpu.make_async_copy(ones_v, acc.at[dstbuf.at[SLAB - 4 + p]],
                                  ssems[p]).wait()
        plsc.subcore_barrier()
        pltpu.sync_copy(acc.at[pl.ds(s * ZR, ZR)],
                        out_hbm.at[c, pl.ds(s * ZR, ZR)])

    return deg_kernel


QD = 5


def _make_gs_kernel():
    @functools.partial(
        pl.kernel,
        out_type=jax.ShapeDtypeStruct((2, NP, HH), jnp.float32),
        mesh=_sc_mesh(),
        compiler_params=pltpu.CompilerParams(use_tc_tiling_on_sc=False),
        scratch_types=(
            [pltpu.VMEM((128,), jnp.int32)] * QD
            + [pltpu.VMEM((SLAB, 128), jnp.int32)]
            + [pltpu.VMEM((128, HH), jnp.float32)] * QD
            + [pltpu.SemaphoreType.DMA] * (3 * QD)
            + [pltpu.VMEM_SHARED((NP, HH), jnp.float32)]
        ),
    )
    def gs_kernel(src_hbm, dst_hbm, hs_hbm, zeros_hbm, out_hbm, *refs):
        idxs = refs[0:QD]
        dstbuf = refs[QD]
        rows = refs[QD + 1:2 * QD + 1]
        isems = refs[2 * QD + 1:3 * QD + 1]
        gsems = refs[3 * QD + 1:4 * QD + 1]
        ssems = refs[4 * QD + 1:5 * QD + 1]
        acc = refs[5 * QD + 1]
        c = lax.axis_index("c")
        s = lax.axis_index("s")
        tile_base = (c * 16 + s) * (LCH * 128)
        pltpu.sync_copy(zeros_hbm, acc.at[pl.ds(s * ZR, ZR)])
        plsc.subcore_barrier()

        def slab_body(k, carry):
            pltpu.sync_copy(dst_hbm.at[s, pl.ds(k * SLAB, SLAB)], dstbuf)

            def group_body(g, carry2):
                not_first = jnp.logical_or(k > 0, g > 0)
                for p in range(QD):
                    jj = g * QD + p
                    base = tile_base + (k * SLAB + jj) * 128

                    @pl.when(not_first)
                    def _():
                        pltpu.make_async_copy(
                            rows[p], acc.at[dstbuf.at[jj]], ssems[p]).wait()

                    pltpu.async_copy(src_hbm.at[pl.ds(base, 128)], idxs[p],
                                     isems[p])
                for p in range(QD):
                    base = tile_base + (g * QD + p) * 128
                    pltpu.make_async_copy(src_hbm.at[pl.ds(base, 128)],
                                          idxs[p], isems[p]).wait()
                    pltpu.async_copy(hs_hbm.at[idxs[p]], rows[p], gsems[p])
                for p in range(QD):
                    jj = g * QD + p
                    pltpu.make_async_copy(hs_hbm.at[idxs[p]], rows[p],
                                          gsems[p]).wait()
                    pltpu.async_copy(rows[p], acc.at[dstbuf.at[jj]],
                                     ssems[p], add=True)
                return carry2

            return lax.fori_loop(0, SLAB // QD, group_body, carry)

        lax.fori_loop(0, LCH // SLAB, slab_body, 0)
        for p in range(QD):
            pltpu.make_async_copy(rows[p], acc.at[dstbuf.at[SLAB - QD + p]],
                                  ssems[p]).wait()
        plsc.subcore_barrier()
        pltpu.sync_copy(acc.at[pl.ds(s * ZR, ZR)],
                        out_hbm.at[c, pl.ds(s * ZR, ZR)])

    return gs_kernel


def _enc_body(x_ref, p_ref, Wn1_ref, bn1_ref, Wn2_ref, bn2_ref, Wc_ref,
              h_ref, dinv_ref, hs_ref, st_ref, st_acc):
    j = pl.program_id(0)
    x = x_ref[...]
    h1 = jnp.maximum(jnp.dot(x, Wn1_ref[...],
                             preferred_element_type=jnp.float32)
                     + bn1_ref[...], 0.0)
    h = jnp.maximum(jnp.dot(h1, Wn2_ref[...],
                            preferred_element_type=jnp.float32)
                    + bn2_ref[...], 0.0)
    deg = 1.0 + p_ref[0, :, 0:1] + p_ref[1, :, 0:1]
    dinv = lax.rsqrt(deg)
    h_ref[...] = h
    dinv_ref[...] = dinv
    hs = jnp.dot(h, Wc_ref[...], preferred_element_type=jnp.float32) * dinv
    hs_ref[0] = hs[:, :HH]
    hs_ref[1] = hs[:, HH:]
    rid = lax.broadcasted_iota(jnp.int32, (BR, 1), 0) + j * BR
    ms = jnp.min(jnp.where(x[:, 2:3] == 1.0, rid, N))
    mt = jnp.min(jnp.where(x[:, 3:4] == 1.0, rid, N))

    @pl.when(j == 0)
    def _():
        st_acc[0] = N
        st_acc[1] = N

    st_acc[0] = jnp.minimum(st_acc[0], ms)
    st_acc[1] = jnp.minimum(st_acc[1], mt)

    @pl.when(j == NB - 1)
    def _():
        lane = lax.broadcasted_iota(jnp.int32, (1, 8), 1)
        st_ref[...] = jnp.where(lane == 0, st_acc[0],
                                jnp.where(lane == 1, st_acc[1], 0))


def _enc_call(x, p, Wn1, bn1, Wn2, bn2, Wc):
    return pl.pallas_call(
        _enc_body,
        grid=(NB,),
        in_specs=[
            pl.BlockSpec((BR, 4), lambda j: (j, 0)),
            pl.BlockSpec((2, BR, 16), lambda j: (0, j, 0)),
            pl.BlockSpec((4, H), lambda j: (0, 0)),
            pl.BlockSpec((1, H), lambda j: (0, 0)),
            pl.BlockSpec((H, H), lambda j: (0, 0)),
            pl.BlockSpec((1, H), lambda j: (0, 0)),
            pl.BlockSpec((H, H), lambda j: (0, 0)),
        ],
        out_specs=[
            pl.BlockSpec((BR, H), lambda j: (j, 0)),
            pl.BlockSpec((BR, 1), lambda j: (j, 0)),
            pl.BlockSpec((2, BR, HH), lambda j: (0, j, 0)),
            pl.BlockSpec((1, 8), lambda j: (0, 0)),
        ],
        out_shape=[
            jax.ShapeDtypeStruct((N, H), jnp.float32),
            jax.ShapeDtypeStruct((N, 1), jnp.float32),
            jax.ShapeDtypeStruct((2, N, HH), jnp.float32),
            jax.ShapeDtypeStruct((1, 8), jnp.int32),
        ],
        scratch_shapes=[pltpu.SMEM((2,), jnp.int32)],
    )(x, p, Wn1, bn1, Wn2, bn2, Wc)


def _mid_body(acc_ref, hs_ref, h_ref, dinv_ref, b_ref, W_ref,
              h2_ref, hs2_ref):
    dinv = dinv_ref[...]
    m = jnp.concatenate([acc_ref[0] + hs_ref[0], acc_ref[1] + hs_ref[1]],
                        axis=1)
    h2 = h_ref[...] + jnp.maximum(m * dinv + b_ref[...], 0.0)
    h2_ref[...] = h2
    hs2 = jnp.dot(h2, W_ref[...], preferred_element_type=jnp.float32) * dinv
    hs2_ref[0] = hs2[:, :HH]
    hs2_ref[1] = hs2[:, HH:]


def _mid_call(acc, hs, h, dinv, b, W):
    return pl.pallas_call(
        _mid_body,
        grid=(NB,),
        in_specs=[
            pl.BlockSpec((2, BR, HH), lambda j: (0, j, 0)),
            pl.BlockSpec((2, BR, HH), lambda j: (0, j, 0)),
            pl.BlockSpec((BR, H), lambda j: (j, 0)),
            pl.BlockSpec((BR, 1), lambda j: (j, 0)),
            pl.BlockSpec((1, H), lambda j: (0, 0)),
            pl.BlockSpec((H, H), lambda j: (0, 0)),
        ],
        out_specs=[
            pl.BlockSpec((BR, H), lambda j: (j, 0)),
            pl.BlockSpec((2, BR, HH), lambda j: (0, j, 0)),
        ],
        out_shape=[
            jax.ShapeDtypeStruct((N, H), jnp.float32),
            jax.ShapeDtypeStruct((2, N, HH), jnp.float32),
        ],
    )(acc, hs, h, dinv, b, W)


def _post_body(si_ti_ref, acc_ref, hs_ref, h_ref, dinv_ref, b_ref,
               Wo1_ref, bo1_ref, Wo2_ref, bo2_ref, Wo3_ref, bo3_ref,
               o_ref, sel_acc):
    j = pl.program_id(0)
    m = jnp.concatenate([acc_ref[0] + hs_ref[0], acc_ref[1] + hs_ref[1]],
                        axis=1)
    h3 = h_ref[...] + jnp.maximum(m * dinv_ref[...] + b_ref[...], 0.0)
    rid = lax.broadcasted_iota(jnp.int32, (BR, 1), 0) + j * BR
    sel2 = jnp.concatenate(
        [jnp.where(rid == si_ti_ref[0], 1.0, 0.0),
         jnp.where(rid == si_ti_ref[1], 1.0, 0.0)], axis=1)
    part = lax.dot_general(sel2, h3, (((0,), (0,)), ((), ())),
                           preferred_element_type=jnp.float32)

    @pl.when(j == 0)
    def _():
        sel_acc[...] = jnp.zeros_like(sel_acc)

    sel_acc[...] += part

    @pl.when(j == NB - 1)
    def _():
        combined = jnp.concatenate([sel_acc[0:1, :], sel_acc[1:2, :]], axis=1)
        o = jnp.maximum(jnp.dot(combined, Wo1_ref[...],
                                preferred_element_type=jnp.float32)
                        + bo1_ref[...], 0.0)
        o = jnp.maximum(jnp.dot(o, Wo2_ref[...],
                                preferred_element_type=jnp.float32)
                        + bo2_ref[...], 0.0)
        o_ref[...] = (jnp.dot(o, Wo3_ref[...],
                              preferred_element_type=jnp.float32)
                      + bo3_ref[...])


def _post_call(si_ti, acc, hs, h, dinv, b, Wo1, bo1, Wo2, bo2, Wo3, bo3):
    full = lambda j, *_: (0, 0)
    return pl.pallas_call(
        _post_body,
        grid_spec=pltpu.PrefetchScalarGridSpec(
            num_scalar_prefetch=1,
            grid=(NB,),
            in_specs=[
                pl.BlockSpec((2, BR, HH), lambda j, *_: (0, j, 0)),
                pl.BlockSpec((2, BR, HH), lambda j, *_: (0, j, 0)),
                pl.BlockSpec((BR, H), lambda j, *_: (j, 0)),
                pl.BlockSpec((BR, 1), lambda j, *_: (j, 0)),
                pl.BlockSpec((1, H), full),
                pl.BlockSpec((2 * H, H), full),
                pl.BlockSpec((1, H), full),
                pl.BlockSpec((H, H // 2), full),
                pl.BlockSpec((1, H // 2), full),
                pl.BlockSpec((H // 2, 1), full),
                pl.BlockSpec((1, 1), full),
            ],
            out_specs=pl.BlockSpec((1, 1), full),
            scratch_shapes=[pltpu.VMEM((2, H), jnp.float32)],
        ),
        out_shape=jax.ShapeDtypeStruct((1, 1), jnp.float32),
    )(si_ti, acc, hs, h, dinv, b, Wo1, bo1, Wo2, bo2, Wo3, bo3)


def kernel(x, edge_index, edge_attr, Wn1, bn1, Wn2, bn2, Wc0, bc0, Wc1, bc1,
           Wc2, bc2, Wo1, bo1, Wo2, bo2, Wo3, bo3):
    f32 = jnp.float32
    src = edge_index[0]
    dst = edge_index[1]
    pad = EP - E
    ar = jnp.arange(pad, dtype=jnp.int32)
    srcp = jnp.concatenate([src, (ar * 977) % N])
    dstp = jnp.concatenate([dst, N + (ar % (NP - N))])
    src4 = jnp.concatenate([srcp, srcp + N])
    dst3 = dstp.reshape(16, LCH, 128)
    dstd = dstp.reshape(2, 16, DCH, 128)
    zeros16 = jnp.zeros((ZR, 16), f32)
    zeros32 = jnp.zeros((ZR, HH), f32)
    ones16 = jnp.ones((128, 16), f32)

    deg_k = _make_deg_kernel()
    gs_k = _make_gs_kernel()

    p = deg_k(dstd, zeros16, ones16)

    h, dinv, hs, st = _enc_call(x, p, Wn1, bn1.reshape(1, H), Wn2,
                                bn2.reshape(1, H), Wc0)

    acc = gs_k(src4, dst3, hs.reshape(2 * N, HH), zeros32)
    h, hs = _mid_call(acc, hs, h, dinv, bc0.reshape(1, H), Wc1)

    acc = gs_k(src4, dst3, hs.reshape(2 * N, HH), zeros32)
    h, hs = _mid_call(acc, hs, h, dinv, bc1.reshape(1, H), Wc2)

    acc = gs_k(src4, dst3, hs.reshape(2 * N, HH), zeros32)

    ms, mt = st[0, 0], st[0, 1]
    has_both = jnp.logical_and(ms < N, mt < N)
    si = jnp.where(has_both, ms, 0)
    ti = jnp.where(has_both, mt, N - 1)
    si_ti = jnp.stack([si, ti])

    o = _post_call(si_ti, acc, hs, h, dinv, bc2.reshape(1, H),
                   Wo1, bo1.reshape(1, H), Wo2, bo2.reshape(1, H // 2),
                   Wo3, bo3.reshape(1, 1))
    return o.reshape(1)
